# baseline (speedup 1.0000x reference)
import functools

import jax
import jax.numpy as jnp
from jax import lax
from jax.experimental import pallas as pl
from jax.experimental.pallas import tpu as pltpu
from jax.experimental.pallas import tpu_sc as plsc

H = 64
LANES = 16
E = 4
NBUF = 4
NREP = 512


def _relayout_table(pw):
    V, Hd = pw.shape
    C = 512
    grid = (V + C - 1) // C

    def body(in_ref, eye_ref, out_ref):
        t = jax.lax.dot_general(in_ref[...], eye_ref[...],
                                (((0,), (0,)), ((), ())),
                                preferred_element_type=jnp.float32)
        out_ref[...] = jnp.concatenate([t, t], axis=1)

    return pl.pallas_call(
        body,
        grid=(grid,),
        in_specs=[pl.BlockSpec((Hd, C), lambda i: (0, i)),
                  pl.BlockSpec((Hd, Hd), lambda i: (0, 0))],
        out_specs=pl.BlockSpec((C, 2 * Hd), lambda i: (i, 0)),
        out_shape=jax.ShapeDtypeStruct((V, 2 * Hd), jnp.float32),
    )(pw.T, jnp.eye(Hd, dtype=jnp.float32))


@functools.lru_cache(maxsize=None)
def _make_kernel(B, L, num_cores, num_subcores):
    NW = num_cores * num_subcores
    BW = B // NW
    nch = BW // E
    CR = E * L
    NG = CR // LANES
    assert B % NW == 0 and BW % E == 0 and nch % NBUF == 0 and L == 20

    mesh = plsc.VectorSubcoreMesh(core_axis_name="c", subcore_axis_name="s")

    data_bufs = []
    for _ in range(NBUF):
        data_bufs += [
            pltpu.VMEM((E, L, H), jnp.float32),
            pltpu.VMEM((CR, 2 * H), jnp.float32),
            pltpu.VMEM((CR, H), jnp.float32),
            pltpu.SemaphoreType.DMA,
            pltpu.SemaphoreType.DMA,
        ]

    @functools.partial(
        pl.kernel,
        mesh=mesh,
        compiler_params=pltpu.CompilerParams(use_tc_tiling_on_sc=False,
                                             needs_layout_passes=False),
        out_type=jax.ShapeDtypeStruct((B, L, H), jnp.float32),
        scratch_types=[
            pltpu.VMEM((BW, L), jnp.int32),
            pltpu.VMEM((BW, L), jnp.int32),
            pltpu.VMEM((nch, CR), jnp.int32),
            pltpu.VMEM((nch, CR), jnp.int32),
        ] + data_bufs,
    )
    def k(x_hbm, pid_hbm, act_hbm, pos_hbm, ptab_hbm, ctab_hbm, out_hbm,
          stage_a, stage_b, pid_idx, idx2_idx, *bufs):
        xb = [bufs[5 * b + 0] for b in range(NBUF)]
        pb = [bufs[5 * b + 1] for b in range(NBUF)]
        cb = [bufs[5 * b + 2] for b in range(NBUF)]
        isem = [bufs[5 * b + 3] for b in range(NBUF)]
        osem = [bufs[5 * b + 4] for b in range(NBUF)]

        wid = lax.axis_index("s") * num_cores + lax.axis_index("c")
        bbase = wid * BW

        pltpu.sync_copy(act_hbm.at[pl.ds(bbase, BW)], stage_a)
        pltpu.sync_copy(pos_hbm.at[pl.ds(bbase, BW)], stage_b)

        def fuse_body(i, carry):
            row0 = i * E
            for g in range(NG):
                fv = lax.iota(jnp.int32, LANES) + (g * LANES)
                rpat = fv // L
                cv = fv - rpat * L
                rv = row0 + rpat
                av = plsc.load_gather(stage_a, [rv, cv])
                ov = plsc.load_gather(stage_b, [rv, cv])
                rep = lax.bitwise_and(wid * nch + i, NREP - 1) * 30
                idx2_idx[i, pl.ds(g * LANES, LANES)] = av * 10 + ov + rep
            return carry

        lax.fori_loop(0, nch, fuse_body, 0)
        pltpu.sync_copy(pid_hbm.at[pl.ds(bbase, BW)], stage_a)

        def repack_body(i, carry):
            row0 = i * E
            for g in range(NG):
                fv = lax.iota(jnp.int32, LANES) + (g * LANES)
                rpat = fv // L
                cv = fv - rpat * L
                rv = row0 + rpat
                pv = plsc.load_gather(stage_a, [rv, cv])
                pid_idx[i, pl.ds(g * LANES, LANES)] = pv
            return carry

        lax.fori_loop(0, nch, repack_body, 0)

        def issue_in(i, p):
            boff = bbase + i * E
            pltpu.async_copy(x_hbm.at[pl.ds(boff, E)], xb[p], isem[p])
            pltpu.async_copy(ptab_hbm.at[pid_idx.at[i]], pb[p], isem[p])
            pltpu.async_copy(ctab_hbm.at[idx2_idx.at[i]], cb[p], isem[p])

        def wait_in(i, p):
            boff = bbase + i * E
            pltpu.make_async_copy(x_hbm.at[pl.ds(boff, E)], xb[p], isem[p]).wait()
            pltpu.make_async_copy(ptab_hbm.at[pid_idx.at[i]], pb[p], isem[p]).wait()
            pltpu.make_async_copy(ctab_hbm.at[idx2_idx.at[i]], cb[p], isem[p]).wait()

        def wait_out(p):
            pltpu.make_async_copy(xb[p], out_hbm.at[pl.ds(bbase, E)],
                                  osem[p]).wait()

        for p in range(NBUF - 1):
            issue_in(p, p)

        def step(t, carry):
            for s in range(NBUF):
                i = NBUF * t + s
                p = s
                wait_in(i, p)

                def row_body(r, rc):
                    for e in range(E):
                        q = e * L + r
                        for g in range(H // LANES):
                            sl = pl.ds(g * LANES, LANES)
                            plsc.addupdate(xb[p].at[e, r, sl], pb[p][q, sl])
                            plsc.addupdate(xb[p].at[e, r, sl], cb[p][q, sl])
                    return rc

                lax.fori_loop(0, L, row_body, 0)
                pltpu.async_copy(xb[p], out_hbm.at[pl.ds(bbase + i * E, E)],
                                 osem[p])

                nxt = i + NBUF - 1
                pn = (s + NBUF - 1) % NBUF

                @pl.when(i >= 1)
                def _():
                    wait_out(pn)

                @pl.when(nxt < nch)
                def _():
                    issue_in(nxt, pn)
            return carry

        lax.fori_loop(0, nch // NBUF, step, 0)
        wait_out(NBUF - 1)

    return k


def kernel(x, player_ids, actions, positions, player_weight, action_weight,
           position_weight):
    B, L, Hd = x.shape
    pid = player_ids.astype(jnp.int32)
    act = actions.astype(jnp.int32)
    pos = positions.astype(jnp.int32)
    combo = (action_weight[:, None, :] + position_weight[None, :, :]).reshape(
        -1, Hd)
    combo = jnp.tile(combo, (NREP, 1))
    t128 = _relayout_table(player_weight)
    info = plsc.get_sparse_core_info()
    return _make_kernel(B, L, info.num_cores, info.num_subcores)(
        x, pid, act, pos, t128, combo)


# speedup vs baseline: 1.8851x; 1.8851x over previous
import functools

import jax
import jax.numpy as jnp
from jax import lax
from jax.experimental import pallas as pl
from jax.experimental.pallas import tpu as pltpu
from jax.experimental.pallas import tpu_sc as plsc

H = 64
LANES = 16
E = 4
NBUF = 4
NREP = 512


def _relayout_table(pw):
    V, Hd = pw.shape
    C = 4096
    grid = (V + C - 1) // C

    def body(in_ref, eye_ref, out_ref):
        t = jax.lax.dot_general(in_ref[...], eye_ref[...],
                                (((0,), (0,)), ((), ())),
                                preferred_element_type=jnp.float32)
        out_ref[...] = jnp.concatenate([t, t], axis=1)

    return pl.pallas_call(
        body,
        grid=(grid,),
        in_specs=[pl.BlockSpec((Hd, C), lambda i: (0, i)),
                  pl.BlockSpec((Hd, Hd), lambda i: (0, 0))],
        out_specs=pl.BlockSpec((C, 2 * Hd), lambda i: (i, 0)),
        out_shape=jax.ShapeDtypeStruct((V, 2 * Hd), jnp.float32),
    )(pw.T, jnp.eye(Hd, dtype=jnp.float32))


@functools.lru_cache(maxsize=None)
def _make_kernel(B, L, num_cores, num_subcores):
    NW = num_cores * num_subcores
    BW = B // NW
    nch = BW // E
    CR = E * L
    NG = CR // LANES
    assert B % NW == 0 and BW % E == 0 and nch % NBUF == 0 and L == 20

    mesh = plsc.VectorSubcoreMesh(core_axis_name="c", subcore_axis_name="s")

    data_bufs = []
    for _ in range(NBUF):
        data_bufs += [
            pltpu.VMEM((E, L, H), jnp.float32),
            pltpu.VMEM((CR, 2 * H), jnp.float32),
            pltpu.VMEM((CR, H), jnp.float32),
            pltpu.SemaphoreType.DMA,
            pltpu.SemaphoreType.DMA,
        ]

    @functools.partial(
        pl.kernel,
        mesh=mesh,
        compiler_params=pltpu.CompilerParams(use_tc_tiling_on_sc=False,
                                             needs_layout_passes=False),
        out_type=jax.ShapeDtypeStruct((B, L, H), jnp.float32),
        scratch_types=[
            pltpu.VMEM((BW, L), jnp.int32),
            pltpu.VMEM((BW, L), jnp.int32),
            pltpu.VMEM((nch, CR), jnp.int32),
            pltpu.VMEM((nch, CR), jnp.int32),
        ] + data_bufs,
    )
    def k(x_hbm, pid_hbm, act_hbm, pos_hbm, ptab_hbm, ctab_hbm, out_hbm,
          stage_a, stage_b, pid_idx, idx2_idx, *bufs):
        xb = [bufs[5 * b + 0] for b in range(NBUF)]
        pb = [bufs[5 * b + 1] for b in range(NBUF)]
        cb = [bufs[5 * b + 2] for b in range(NBUF)]
        isem = [bufs[5 * b + 3] for b in range(NBUF)]
        osem = [bufs[5 * b + 4] for b in range(NBUF)]

        wid = lax.axis_index("s") * num_cores + lax.axis_index("c")
        bbase = wid * BW

        pltpu.sync_copy(act_hbm.at[pl.ds(bbase, BW)], stage_a)
        pltpu.sync_copy(pos_hbm.at[pl.ds(bbase, BW)], stage_b)

        def fuse_body(i, carry):
            row0 = i * E
            for g in range(NG):
                fv = lax.iota(jnp.int32, LANES) + (g * LANES)
                rpat = fv // L
                cv = fv - rpat * L
                rv = row0 + rpat
                av = plsc.load_gather(stage_a, [rv, cv])
                ov = plsc.load_gather(stage_b, [rv, cv])
                rep = lax.bitwise_and(wid * nch + i, NREP - 1) * 30
                idx2_idx[i, pl.ds(g * LANES, LANES)] = av * 10 + ov + rep
            return carry

        lax.fori_loop(0, nch, fuse_body, 0)
        pltpu.sync_copy(pid_hbm.at[pl.ds(bbase, BW)], stage_a)

        def repack_body(i, carry):
            row0 = i * E
            for g in range(NG):
                fv = lax.iota(jnp.int32, LANES) + (g * LANES)
                rpat = fv // L
                cv = fv - rpat * L
                rv = row0 + rpat
                pv = plsc.load_gather(stage_a, [rv, cv])
                pid_idx[i, pl.ds(g * LANES, LANES)] = pv
            return carry

        lax.fori_loop(0, nch, repack_body, 0)

        def issue_in(i, p):
            boff = bbase + i * E
            pltpu.async_copy(x_hbm.at[pl.ds(boff, E)], xb[p], isem[p])
            pltpu.async_copy(ptab_hbm.at[pid_idx.at[i]], pb[p], isem[p])
            pltpu.async_copy(ctab_hbm.at[idx2_idx.at[i]], cb[p], isem[p])

        def wait_in(i, p):
            boff = bbase + i * E
            pltpu.make_async_copy(x_hbm.at[pl.ds(boff, E)], xb[p], isem[p]).wait()
            pltpu.make_async_copy(ptab_hbm.at[pid_idx.at[i]], pb[p], isem[p]).wait()
            pltpu.make_async_copy(ctab_hbm.at[idx2_idx.at[i]], cb[p], isem[p]).wait()

        def wait_out(p):
            pltpu.make_async_copy(xb[p], out_hbm.at[pl.ds(bbase, E)],
                                  osem[p]).wait()

        for p in range(NBUF - 1):
            issue_in(p, p)

        def step(t, carry):
            for s in range(NBUF):
                i = NBUF * t + s
                p = s
                wait_in(i, p)

                def row_body(r, rc):
                    for e in range(E):
                        q = e * L + r
                        for g in range(H // LANES):
                            sl = pl.ds(g * LANES, LANES)
                            plsc.addupdate(xb[p].at[e, r, sl], pb[p][q, sl])
                            plsc.addupdate(xb[p].at[e, r, sl], cb[p][q, sl])
                    return rc

                lax.fori_loop(0, L, row_body, 0)
                pltpu.async_copy(xb[p], out_hbm.at[pl.ds(bbase + i * E, E)],
                                 osem[p])

                nxt = i + NBUF - 1
                pn = (s + NBUF - 1) % NBUF

                @pl.when(i >= 1)
                def _():
                    wait_out(pn)

                @pl.when(nxt < nch)
                def _():
                    issue_in(nxt, pn)
            return carry

        lax.fori_loop(0, nch // NBUF, step, 0)
        wait_out(NBUF - 1)

    return k


def kernel(x, player_ids, actions, positions, player_weight, action_weight,
           position_weight):
    B, L, Hd = x.shape
    pid = player_ids.astype(jnp.int32)
    act = actions.astype(jnp.int32)
    pos = positions.astype(jnp.int32)
    combo = (action_weight[:, None, :] + position_weight[None, :, :]).reshape(
        -1, Hd)
    combo = jnp.tile(combo, (NREP, 1))
    t128 = _relayout_table(player_weight)
    info = plsc.get_sparse_core_info()
    return _make_kernel(B, L, info.num_cores, info.num_subcores)(
        x, pid, act, pos, t128, combo)


# speedup vs baseline: 2.0642x; 1.0950x over previous
import functools

import jax
import jax.numpy as jnp
from jax import lax
from jax.experimental import pallas as pl
from jax.experimental.pallas import tpu as pltpu
from jax.experimental.pallas import tpu_sc as plsc

H = 64
LANES = 16
E = 4
NBUF = 4
NREP = 512


def _relayout_table(pw):
    V, Hd = pw.shape
    C = 12800
    grid = (V + C - 1) // C

    def body(in_ref, eye_ref, out_ref):
        t = jax.lax.dot_general(in_ref[...], eye_ref[...],
                                (((0,), (0,)), ((), ())),
                                preferred_element_type=jnp.float32)
        out_ref[...] = jnp.concatenate([t, t], axis=1)

    return pl.pallas_call(
        body,
        grid=(grid,),
        in_specs=[pl.BlockSpec((Hd, C), lambda i: (0, i)),
                  pl.BlockSpec((Hd, Hd), lambda i: (0, 0))],
        out_specs=pl.BlockSpec((C, 2 * Hd), lambda i: (i, 0)),
        out_shape=jax.ShapeDtypeStruct((V, 2 * Hd), jnp.float32),
    )(pw.T, jnp.eye(Hd, dtype=jnp.float32))


@functools.lru_cache(maxsize=None)
def _make_kernel(B, L, num_cores, num_subcores):
    NW = num_cores * num_subcores
    BW = B // NW
    nch = BW // E
    CR = E * L
    NG = CR // LANES
    assert B % NW == 0 and BW % E == 0 and nch % NBUF == 0 and L == 20

    mesh = plsc.VectorSubcoreMesh(core_axis_name="c", subcore_axis_name="s")

    data_bufs = []
    for _ in range(NBUF):
        data_bufs += [
            pltpu.VMEM((E, L, H), jnp.float32),
            pltpu.VMEM((CR, 2 * H), jnp.float32),
            pltpu.VMEM((CR, H), jnp.float32),
            pltpu.SemaphoreType.DMA,
            pltpu.SemaphoreType.DMA,
        ]

    @functools.partial(
        pl.kernel,
        mesh=mesh,
        compiler_params=pltpu.CompilerParams(use_tc_tiling_on_sc=False,
                                             needs_layout_passes=False),
        out_type=jax.ShapeDtypeStruct((B, L, H), jnp.float32),
        scratch_types=[
            pltpu.VMEM((BW, L), jnp.int32),
            pltpu.VMEM((BW, L), jnp.int32),
            pltpu.VMEM((nch, CR), jnp.int32),
            pltpu.VMEM((nch, CR), jnp.int32),
        ] + data_bufs,
    )
    def k(x_hbm, pid_hbm, act_hbm, pos_hbm, ptab_hbm, ctab_hbm, out_hbm,
          stage_a, stage_b, pid_idx, idx2_idx, *bufs):
        xb = [bufs[5 * b + 0] for b in range(NBUF)]
        pb = [bufs[5 * b + 1] for b in range(NBUF)]
        cb = [bufs[5 * b + 2] for b in range(NBUF)]
        isem = [bufs[5 * b + 3] for b in range(NBUF)]
        osem = [bufs[5 * b + 4] for b in range(NBUF)]

        wid = lax.axis_index("s") * num_cores + lax.axis_index("c")
        bbase = wid * BW

        pltpu.sync_copy(act_hbm.at[pl.ds(bbase, BW)], stage_a)
        pltpu.sync_copy(pos_hbm.at[pl.ds(bbase, BW)], stage_b)

        def fuse_body(i, carry):
            row0 = i * E
            for g in range(NG):
                fv = lax.iota(jnp.int32, LANES) + (g * LANES)
                rpat = fv // L
                cv = fv - rpat * L
                rv = row0 + rpat
                av = plsc.load_gather(stage_a, [rv, cv])
                ov = plsc.load_gather(stage_b, [rv, cv])
                rep = lax.bitwise_and(wid * nch + i, NREP - 1) * 30
                idx2_idx[i, pl.ds(g * LANES, LANES)] = av * 10 + ov + rep
            return carry

        lax.fori_loop(0, nch, fuse_body, 0)
        pltpu.sync_copy(pid_hbm.at[pl.ds(bbase, BW)], stage_a)

        def repack_body(i, carry):
            row0 = i * E
            for g in range(NG):
                fv = lax.iota(jnp.int32, LANES) + (g * LANES)
                rpat = fv // L
                cv = fv - rpat * L
                rv = row0 + rpat
                pv = plsc.load_gather(stage_a, [rv, cv])
                pid_idx[i, pl.ds(g * LANES, LANES)] = pv
            return carry

        lax.fori_loop(0, nch, repack_body, 0)

        def issue_in(i, p):
            boff = bbase + i * E
            pltpu.async_copy(x_hbm.at[pl.ds(boff, E)], xb[p], isem[p])
            pltpu.async_copy(ptab_hbm.at[pid_idx.at[i]], pb[p], isem[p])
            pltpu.async_copy(ctab_hbm.at[idx2_idx.at[i]], cb[p], isem[p])

        def wait_in(i, p):
            boff = bbase + i * E
            pltpu.make_async_copy(x_hbm.at[pl.ds(boff, E)], xb[p], isem[p]).wait()
            pltpu.make_async_copy(ptab_hbm.at[pid_idx.at[i]], pb[p], isem[p]).wait()
            pltpu.make_async_copy(ctab_hbm.at[idx2_idx.at[i]], cb[p], isem[p]).wait()

        def wait_out(p):
            pltpu.make_async_copy(xb[p], out_hbm.at[pl.ds(bbase, E)],
                                  osem[p]).wait()

        for p in range(NBUF - 1):
            issue_in(p, p)

        def step(t, carry):
            for s in range(NBUF):
                i = NBUF * t + s
                p = s
                wait_in(i, p)

                def row_body(r, rc):
                    for e in range(E):
                        q = e * L + r
                        for g in range(H // LANES):
                            sl = pl.ds(g * LANES, LANES)
                            plsc.addupdate(xb[p].at[e, r, sl], pb[p][q, sl])
                            plsc.addupdate(xb[p].at[e, r, sl], cb[p][q, sl])
                    return rc

                lax.fori_loop(0, L, row_body, 0)
                pltpu.async_copy(xb[p], out_hbm.at[pl.ds(bbase + i * E, E)],
                                 osem[p])

                nxt = i + NBUF - 1
                pn = (s + NBUF - 1) % NBUF

                @pl.when(i >= 1)
                def _():
                    wait_out(pn)

                @pl.when(nxt < nch)
                def _():
                    issue_in(nxt, pn)
            return carry

        lax.fori_loop(0, nch // NBUF, step, 0)
        wait_out(NBUF - 1)

    return k


def kernel(x, player_ids, actions, positions, player_weight, action_weight,
           position_weight):
    B, L, Hd = x.shape
    pid = player_ids.astype(jnp.int32)
    act = actions.astype(jnp.int32)
    pos = positions.astype(jnp.int32)
    combo = (action_weight[:, None, :] + position_weight[None, :, :]).reshape(
        -1, Hd)
    combo = jnp.tile(combo, (NREP, 1))
    t128 = _relayout_table(player_weight)
    info = plsc.get_sparse_core_info()
    return _make_kernel(B, L, info.num_cores, info.num_subcores)(
        x, pid, act, pos, t128, combo)
